# Initial kernel scaffold; baseline (speedup 1.0000x reference)
import jax
import jax.numpy as jnp
from jax.experimental import pallas as pl


def kernel(x, edge_index, edge_attr, batch, params):
    raise NotImplementedError("write your pallas kernel here")


# speedup vs baseline: 5.0115x; 5.0115x over previous
import functools

import jax
import jax.numpy as jnp
from jax import lax
from jax.experimental import pallas as pl
from jax.experimental.pallas import tpu as pltpu
from jax.experimental.pallas import tpu_sc as plsc

_N = 10000
_E = 320000
_NG = 64
_EPS = 1e-5
_R = 400
_CH = 128
_D = 128
_HIGH = jax.lax.Precision.HIGHEST


def _dot(a, b):
    return jax.lax.dot_general(a, b, (((1,), (0,)), ((), ())),
                               precision=_HIGH,
                               preferred_element_type=jnp.float32)


@functools.lru_cache(maxsize=None)
def _agg_call(interpret=False):
    mesh = plsc.VectorSubcoreMesh(core_axis_name="c", subcore_axis_name="s",
                                  num_cores=2, num_subcores=16)
    NCC = (_E // _CH) // 2
    RT = 624
    RZ = 78

    def body(h, src, dst, a0, a1, aggS, sbuf, dbuf, rows, zbuf, sem):
        c = lax.axis_index("c")
        s = lax.axis_index("s")

        def work(aout, base):
            def zi(i, _):
                def zj(j, __):
                    zbuf[i, pl.ds(j * 16, 16)] = jnp.zeros((16,), jnp.float32)
                    return 0
                return lax.fori_loop(0, _D // 16, zj, 0)
            lax.fori_loop(0, RZ, zi, 0)
            for k in range(8):
                pltpu.sync_copy(zbuf, aggS.at[pl.ds(s * RT + k * RZ, RZ)])
            pl.when(s == 0)(lambda: pltpu.sync_copy(
                zbuf.at[pl.ds(0, 16)], aggS.at[pl.ds(16 * RT, 16)]))
            plsc.subcore_barrier()

            lo = base + (s * NCC) // 16
            hi = base + ((s + 1) * NCC) // 16

            def eb(g, _):
                pltpu.sync_copy(src.at[pl.ds(g * _CH, _CH)], sbuf)
                pltpu.sync_copy(dst.at[pl.ds(g * _CH, _CH)], dbuf)
                pltpu.async_copy(h.at[sbuf], rows, sem).wait()
                pltpu.sync_copy(rows, aggS.at[dbuf], add=True)
                return 0
            lax.fori_loop(lo, hi, eb, 0)
            plsc.subcore_barrier()
            pltpu.sync_copy(aggS.at[pl.ds(s * RT, RT)],
                            aout.at[pl.ds(s * RT, RT)])
            pl.when(s == 0)(lambda: pltpu.sync_copy(
                aggS.at[pl.ds(16 * RT, 16)], aout.at[pl.ds(16 * RT, 16)]))

        pl.when(c == 0)(lambda: work(a0, 0))
        pl.when(c == 1)(lambda: work(a1, NCC))

    out = (jax.ShapeDtypeStruct((_N, _D), jnp.float32),
           jax.ShapeDtypeStruct((_N, _D), jnp.float32))
    return pl.kernel(
        body, out_type=out, mesh=mesh,
        scratch_types=[
            pltpu.VMEM_SHARED((_N, _D), jnp.float32),
            pltpu.VMEM((_CH,), jnp.int32),
            pltpu.VMEM((_CH,), jnp.int32),
            pltpu.VMEM((_CH, _D), jnp.float32),
            pltpu.VMEM((RZ, _D), jnp.float32),
            pltpu.SemaphoreType.DMA,
        ],
        interpret=interpret)


@functools.lru_cache(maxsize=None)
def _conv_call(nin, nout, dr, interpret=False):
    NB = _N // _R

    def body(*refs):
        xs = refs[:nin]
        ps = refs[nin:3 * nin]
        w1, b1, w2, b2 = refs[3 * nin:3 * nin + 4]
        outs = refs[3 * nin + 4:3 * nin + 4 + nout]
        st = refs[3 * nin + 4 + nout]
        i = pl.program_id(0)

        h = b1[...]
        for k in range(nin):
            z = xs[k][...] + ps[2 * k][...] + ps[2 * k + 1][...]
            h = h + _dot(z, w1[128 * k:128 * (k + 1), :])
        h = jnp.maximum(h, 0.0)
        h = _dot(h, w2[...]) + b2[...]
        h = jnp.maximum(h, 0.0)
        for k in range(nout):
            outs[k][...] = h[:, 128 * k:128 * (k + 1)]

        @pl.when(i == 0)
        def _():
            st[...] = jnp.zeros_like(st)
        s1 = jnp.sum(h, axis=0)[None, :]
        s2 = jnp.sum(h * h, axis=0)[None, :]
        st[...] += jnp.concatenate([s1, s2], axis=0)

    slab = pl.BlockSpec((_R, _D), lambda i: (i, 0))
    return pl.pallas_call(
        body,
        grid=(NB,),
        in_specs=[slab] * (3 * nin) + [
            pl.BlockSpec((128 * nin, dr), lambda i: (0, 0)),
            pl.BlockSpec((1, dr), lambda i: (0, 0)),
            pl.BlockSpec((dr, 128 * nout), lambda i: (0, 0)),
            pl.BlockSpec((1, 128 * nout), lambda i: (0, 0)),
        ],
        out_specs=[slab] * nout + [pl.BlockSpec((2, 128 * nout),
                                                lambda i: (0, 0))],
        out_shape=[jax.ShapeDtypeStruct((_N, _D), jnp.float32)] * nout
        + [jax.ShapeDtypeStruct((2, 128 * nout), jnp.float32)],
        interpret=interpret)


@functools.lru_cache(maxsize=None)
def _bn_call(nout, interpret=False):
    NB = _N // _R

    def body(*refs):
        hs = refs[:nout]
        stref, g, b = refs[nout:nout + 3]
        outs = refs[nout + 3:]
        st = stref[...]
        mean = st[0:1, :] / _N
        var = st[1:2, :] / _N - mean * mean
        scale = g[...] * jax.lax.rsqrt(var + _EPS)
        shift = b[...] - mean * scale
        for k in range(nout):
            sl = slice(128 * k, 128 * (k + 1))
            outs[k][...] = hs[k][...] * scale[:, sl] + shift[:, sl]

    slab = pl.BlockSpec((_R, _D), lambda i: (i, 0))
    wide = pl.BlockSpec((2, 128 * nout), lambda i: (0, 0))
    row = pl.BlockSpec((1, 128 * nout), lambda i: (0, 0))
    return pl.pallas_call(
        body,
        grid=(NB,),
        in_specs=[slab] * nout + [wide, row, row],
        out_specs=[slab] * nout,
        out_shape=[jax.ShapeDtypeStruct((_N, _D), jnp.float32)] * nout,
        interpret=interpret)


@functools.lru_cache(maxsize=None)
def _pool_call(interpret=False):
    NB = _N // _R

    def body(h, bref, w1, b1, w2, b2, out, acc):
        i = pl.program_id(0)

        @pl.when(i == 0)
        def _():
            acc[...] = jnp.zeros_like(acc)
        ids = lax.broadcasted_iota(jnp.int32, (_NG, _R), 0)
        oht = (ids == bref[...].reshape(1, _R)).astype(jnp.float32)
        acc[...] += _dot(oht, h[...][:, :64])

        @pl.when(i == NB - 1)
        def _():
            g = jnp.maximum(_dot(acc[...], w1[...]) + b1[...], 0.0)
            g = jnp.maximum(_dot(g, w2[...]) + b2[...], 0.0)
            out[...] = g

    return pl.pallas_call(
        body,
        grid=(NB,),
        in_specs=[
            pl.BlockSpec((_R, _D), lambda i: (i, 0)),
            pl.BlockSpec((1, 1, _R), lambda i: (i, 0, 0)),
            pl.BlockSpec((64, 64), lambda i: (0, 0)),
            pl.BlockSpec((1, 64), lambda i: (0, 0)),
            pl.BlockSpec((64, 1), lambda i: (0, 0)),
            pl.BlockSpec((1, 1), lambda i: (0, 0)),
        ],
        out_specs=pl.BlockSpec((_NG, 1), lambda i: (0, 0)),
        out_shape=jax.ShapeDtypeStruct((_NG, 1), jnp.float32),
        scratch_shapes=[pltpu.VMEM((_NG, 64), jnp.float32)],
        interpret=interpret)


def _pad_cols(a, w):
    return a if a.shape[-1] == w else jnp.pad(a, [(0, 0)] * (a.ndim - 1)
                                              + [(0, w - a.shape[-1])])


def kernel(x, edge_index, edge_attr, batch, params):
    del edge_attr
    src = edge_index[0]
    dst = edge_index[1]
    batch3 = batch.reshape(_N // _R, 1, _R)
    slabs = [x]

    for convs, bn in zip(params["gins"], params["bns"]):
        nout = 1
        for p in convs:
            din, dout = p["W1"].shape
            nin = len(slabs)
            nout = 2 if dout == 256 else 1
            parts = []
            for sl in slabs:
                parts.extend(_agg_call()(sl, src, dst))
            w1p = jnp.pad(p["W1"], ((0, 128 * nin - din), (0, 0)))
            w2p = _pad_cols(p["W2"], 128 * nout)
            b2p = _pad_cols(p["b2"].reshape(1, dout), 128 * nout)
            res = _conv_call(nin, nout, dout)(
                *slabs, *parts, w1p, p["b1"].reshape(1, dout), w2p, b2p)
            slabs, stats = list(res[:nout]), res[nout]
        gp = _pad_cols(bn["gamma"].reshape(1, -1), 128 * nout)
        bp = _pad_cols(bn["beta"].reshape(1, -1), 128 * nout)
        slabs = list(_bn_call(nout)(*slabs, stats, gp, bp))

    fc1, fc2 = params["fc"]
    return _pool_call()(slabs[0], batch3,
                        fc1["W"], fc1["b"].reshape(1, 64),
                        fc2["W"], fc2["b"].reshape(1, 1))


# speedup vs baseline: 7.7256x; 1.5416x over previous
import functools

import jax
import jax.numpy as jnp
from jax import lax
from jax.experimental import pallas as pl
from jax.experimental.pallas import tpu as pltpu
from jax.experimental.pallas import tpu_sc as plsc

_N = 10000
_E = 320000
_NG = 64
_EPS = 1e-5
_R = 400
_CH = 128
_D = 128
_HIGH = jax.lax.Precision.HIGHEST


def _dot(a, b):
    return jax.lax.dot_general(a, b, (((1,), (0,)), ((), ())),
                               precision=_HIGH,
                               preferred_element_type=jnp.float32)


@functools.lru_cache(maxsize=None)
def _agg_call(interpret=False):
    mesh = plsc.VectorSubcoreMesh(core_axis_name="c", subcore_axis_name="s",
                                  num_cores=2, num_subcores=16)
    NCC = (_E // _CH) // 2
    RT = 624
    RZ = 78

    NBUF = 2

    def body(h, src, dst, a0, a1, aggS, *scr):
        sb = scr[0:NBUF]
        db = scr[NBUF:2 * NBUF]
        rw = scr[2 * NBUF:3 * NBUF]
        zbuf = scr[3 * NBUF]
        gs = scr[3 * NBUF + 1:4 * NBUF + 1]
        ss = scr[4 * NBUF + 1:5 * NBUF + 1]
        c = lax.axis_index("c")
        s = lax.axis_index("s")

        def work(aout, base):
            def zi(i, _):
                def zj(j, __):
                    zbuf[i, pl.ds(j * 16, 16)] = jnp.zeros((16,), jnp.float32)
                    return 0
                return lax.fori_loop(0, _D // 16, zj, 0)
            lax.fori_loop(0, RZ, zi, 0)
            for k in range(8):
                pltpu.sync_copy(zbuf, aggS.at[pl.ds(s * RT + k * RZ, RZ)])
            pl.when(s == 0)(lambda: pltpu.sync_copy(
                zbuf.at[pl.ds(0, 16)], aggS.at[pl.ds(16 * RT, 16)]))
            plsc.subcore_barrier()

            lo = base + (s * NCC) // 16
            hi = base + ((s + 1) * NCC) // 16
            n = hi - lo

            def step(i, b):
                g = lo + i

                @pl.when(i < n)
                def _issue():
                    @pl.when(i >= NBUF)
                    def _():
                        pltpu.make_async_copy(rw[b], aggS.at[db[b]],
                                              ss[b]).wait()
                    pltpu.sync_copy(src.at[pl.ds(g * _CH, _CH)], sb[b])
                    pltpu.sync_copy(dst.at[pl.ds(g * _CH, _CH)], db[b])
                    pltpu.async_copy(h.at[sb[b]], rw[b], gs[b])

                j = i - 1
                bc = (b - 1) % NBUF

                @pl.when((j >= 0) & (j < n))
                def _consume():
                    pltpu.make_async_copy(h.at[sb[bc]], rw[bc], gs[bc]).wait()
                    pltpu.async_copy(rw[bc], aggS.at[db[bc]], ss[bc],
                                     add=True)

            def outer(o, _):
                for b in range(NBUF):
                    step(o * NBUF + b, b)
                return 0
            lax.fori_loop(0, (n + 1 + NBUF - 1) // NBUF, outer, 0)
            for b in range(NBUF):
                pltpu.make_async_copy(rw[b], aggS.at[db[b]], ss[b]).wait()
            plsc.subcore_barrier()
            pltpu.sync_copy(aggS.at[pl.ds(s * RT, RT)],
                            aout.at[pl.ds(s * RT, RT)])
            pl.when(s == 0)(lambda: pltpu.sync_copy(
                aggS.at[pl.ds(16 * RT, 16)], aout.at[pl.ds(16 * RT, 16)]))

        pl.when(c == 0)(lambda: work(a0, 0))
        pl.when(c == 1)(lambda: work(a1, NCC))

    out = (jax.ShapeDtypeStruct((_N, _D), jnp.float32),
           jax.ShapeDtypeStruct((_N, _D), jnp.float32))
    return pl.kernel(
        body, out_type=out, mesh=mesh,
        scratch_types=[pltpu.VMEM_SHARED((_N, _D), jnp.float32)]
        + [pltpu.VMEM((_CH,), jnp.int32)] * (2 * NBUF)
        + [pltpu.VMEM((_CH, _D), jnp.float32)] * NBUF
        + [pltpu.VMEM((RZ, _D), jnp.float32)]
        + [pltpu.SemaphoreType.DMA] * (2 * NBUF),
        interpret=interpret)


@functools.lru_cache(maxsize=None)
def _conv_call(nin, nout, dr, interpret=False):
    NB = _N // _R

    def body(*refs):
        xs = refs[:nin]
        ps = refs[nin:3 * nin]
        w1, b1, w2, b2 = refs[3 * nin:3 * nin + 4]
        outs = refs[3 * nin + 4:3 * nin + 4 + nout]
        st = refs[3 * nin + 4 + nout]
        i = pl.program_id(0)

        h = b1[...]
        for k in range(nin):
            z = xs[k][...] + ps[2 * k][...] + ps[2 * k + 1][...]
            h = h + _dot(z, w1[128 * k:128 * (k + 1), :])
        h = jnp.maximum(h, 0.0)
        h = _dot(h, w2[...]) + b2[...]
        h = jnp.maximum(h, 0.0)
        for k in range(nout):
            outs[k][...] = h[:, 128 * k:128 * (k + 1)]

        @pl.when(i == 0)
        def _():
            st[...] = jnp.zeros_like(st)
        s1 = jnp.sum(h, axis=0)[None, :]
        s2 = jnp.sum(h * h, axis=0)[None, :]
        st[...] += jnp.concatenate([s1, s2], axis=0)

    slab = pl.BlockSpec((_R, _D), lambda i: (i, 0))
    return pl.pallas_call(
        body,
        grid=(NB,),
        in_specs=[slab] * (3 * nin) + [
            pl.BlockSpec((128 * nin, dr), lambda i: (0, 0)),
            pl.BlockSpec((1, dr), lambda i: (0, 0)),
            pl.BlockSpec((dr, 128 * nout), lambda i: (0, 0)),
            pl.BlockSpec((1, 128 * nout), lambda i: (0, 0)),
        ],
        out_specs=[slab] * nout + [pl.BlockSpec((2, 128 * nout),
                                                lambda i: (0, 0))],
        out_shape=[jax.ShapeDtypeStruct((_N, _D), jnp.float32)] * nout
        + [jax.ShapeDtypeStruct((2, 128 * nout), jnp.float32)],
        interpret=interpret)


@functools.lru_cache(maxsize=None)
def _bn_call(nout, interpret=False):
    NB = _N // _R

    def body(*refs):
        hs = refs[:nout]
        stref, g, b = refs[nout:nout + 3]
        outs = refs[nout + 3:]
        st = stref[...]
        mean = st[0:1, :] / _N
        var = st[1:2, :] / _N - mean * mean
        scale = g[...] * jax.lax.rsqrt(var + _EPS)
        shift = b[...] - mean * scale
        for k in range(nout):
            sl = slice(128 * k, 128 * (k + 1))
            outs[k][...] = hs[k][...] * scale[:, sl] + shift[:, sl]

    slab = pl.BlockSpec((_R, _D), lambda i: (i, 0))
    wide = pl.BlockSpec((2, 128 * nout), lambda i: (0, 0))
    row = pl.BlockSpec((1, 128 * nout), lambda i: (0, 0))
    return pl.pallas_call(
        body,
        grid=(NB,),
        in_specs=[slab] * nout + [wide, row, row],
        out_specs=[slab] * nout,
        out_shape=[jax.ShapeDtypeStruct((_N, _D), jnp.float32)] * nout,
        interpret=interpret)


@functools.lru_cache(maxsize=None)
def _pool_call(interpret=False):
    NB = _N // _R

    def body(h, bref, w1, b1, w2, b2, out, acc):
        i = pl.program_id(0)

        @pl.when(i == 0)
        def _():
            acc[...] = jnp.zeros_like(acc)
        ids = lax.broadcasted_iota(jnp.int32, (_NG, _R), 0)
        oht = (ids == bref[...].reshape(1, _R)).astype(jnp.float32)
        acc[...] += _dot(oht, h[...][:, :64])

        @pl.when(i == NB - 1)
        def _():
            g = jnp.maximum(_dot(acc[...], w1[...]) + b1[...], 0.0)
            g = jnp.maximum(_dot(g, w2[...]) + b2[...], 0.0)
            out[...] = g

    return pl.pallas_call(
        body,
        grid=(NB,),
        in_specs=[
            pl.BlockSpec((_R, _D), lambda i: (i, 0)),
            pl.BlockSpec((1, 1, _R), lambda i: (i, 0, 0)),
            pl.BlockSpec((64, 64), lambda i: (0, 0)),
            pl.BlockSpec((1, 64), lambda i: (0, 0)),
            pl.BlockSpec((64, 1), lambda i: (0, 0)),
            pl.BlockSpec((1, 1), lambda i: (0, 0)),
        ],
        out_specs=pl.BlockSpec((_NG, 1), lambda i: (0, 0)),
        out_shape=jax.ShapeDtypeStruct((_NG, 1), jnp.float32),
        scratch_shapes=[pltpu.VMEM((_NG, 64), jnp.float32)],
        interpret=interpret)


def _pad_cols(a, w):
    return a if a.shape[-1] == w else jnp.pad(a, [(0, 0)] * (a.ndim - 1)
                                              + [(0, w - a.shape[-1])])


def kernel(x, edge_index, edge_attr, batch, params):
    del edge_attr
    src = edge_index[0]
    dst = edge_index[1]
    batch3 = batch.reshape(_N // _R, 1, _R)
    slabs = [x]

    for convs, bn in zip(params["gins"], params["bns"]):
        nout = 1
        for p in convs:
            din, dout = p["W1"].shape
            nin = len(slabs)
            nout = 2 if dout == 256 else 1
            parts = []
            for sl in slabs:
                parts.extend(_agg_call()(sl, src, dst))
            w1p = jnp.pad(p["W1"], ((0, 128 * nin - din), (0, 0)))
            w2p = _pad_cols(p["W2"], 128 * nout)
            b2p = _pad_cols(p["b2"].reshape(1, dout), 128 * nout)
            res = _conv_call(nin, nout, dout)(
                *slabs, *parts, w1p, p["b1"].reshape(1, dout), w2p, b2p)
            slabs, stats = list(res[:nout]), res[nout]
        gp = _pad_cols(bn["gamma"].reshape(1, -1), 128 * nout)
        bp = _pad_cols(bn["beta"].reshape(1, -1), 128 * nout)
        slabs = list(_bn_call(nout)(*slabs, stats, gp, bp))

    fc1, fc2 = params["fc"]
    return _pool_call()(slabs[0], batch3,
                        fc1["W"], fc1["b"].reshape(1, 64),
                        fc2["W"], fc2["b"].reshape(1, 1))


# speedup vs baseline: 7.7271x; 1.0002x over previous
import functools

import jax
import jax.numpy as jnp
from jax import lax
from jax.experimental import pallas as pl
from jax.experimental.pallas import tpu as pltpu
from jax.experimental.pallas import tpu_sc as plsc

_N = 10000
_E = 320000
_NG = 64
_EPS = 1e-5
_R = 400
_CH = 128
_D = 128
_HIGH = jax.lax.Precision.HIGHEST


def _dot(a, b):
    return jax.lax.dot_general(a, b, (((1,), (0,)), ((), ())),
                               precision=_HIGH,
                               preferred_element_type=jnp.float32)


@functools.lru_cache(maxsize=None)
def _agg_call(interpret=False):
    mesh = plsc.VectorSubcoreMesh(core_axis_name="c", subcore_axis_name="s",
                                  num_cores=2, num_subcores=16)
    NCC = (_E // _CH) // 2
    RT = 624
    RZ = 78

    NBUF = 2

    def body(h, src, dst, a0, a1, aggS, *scr):
        sb = scr[0:NBUF]
        db = scr[NBUF:2 * NBUF]
        rw = scr[2 * NBUF:3 * NBUF]
        zbuf = scr[3 * NBUF]
        gs = scr[3 * NBUF + 1:4 * NBUF + 1]
        ss = scr[4 * NBUF + 1:5 * NBUF + 1]
        c = lax.axis_index("c")
        s = lax.axis_index("s")

        def work(aout, base):
            def zi(i, _):
                def zj(j, __):
                    zbuf[i, pl.ds(j * 16, 16)] = jnp.zeros((16,), jnp.float32)
                    return 0
                return lax.fori_loop(0, _D // 16, zj, 0)
            lax.fori_loop(0, RZ, zi, 0)
            for k in range(8):
                pltpu.sync_copy(zbuf, aggS.at[pl.ds(s * RT + k * RZ, RZ)])
            pl.when(s == 0)(lambda: pltpu.sync_copy(
                zbuf.at[pl.ds(0, 16)], aggS.at[pl.ds(16 * RT, 16)]))
            plsc.subcore_barrier()

            lo = base + (s * NCC) // 16
            hi = base + ((s + 1) * NCC) // 16
            n = hi - lo

            def step(i, b):
                g = lo + i

                @pl.when(i < n)
                def _issue():
                    pltpu.sync_copy(src.at[pl.ds(g * _CH, _CH)], sb[b])
                    pltpu.sync_copy(dst.at[pl.ds(g * _CH, _CH)], db[b])
                    pltpu.async_copy(h.at[sb[b]], rw[b], gs[b])

                j = i - 1
                bc = (b - 1) % NBUF

                @pl.when((j >= 0) & (j < n))
                def _consume():
                    pltpu.make_async_copy(h.at[sb[bc]], rw[bc], gs[bc]).wait()
                    pltpu.sync_copy(rw[bc], aggS.at[db[bc]], add=True)

            def outer(o, _):
                for b in range(NBUF):
                    step(o * NBUF + b, b)
                return 0
            lax.fori_loop(0, (n + 1 + NBUF - 1) // NBUF, outer, 0)
            plsc.subcore_barrier()
            pltpu.sync_copy(aggS.at[pl.ds(s * RT, RT)],
                            aout.at[pl.ds(s * RT, RT)])
            pl.when(s == 0)(lambda: pltpu.sync_copy(
                aggS.at[pl.ds(16 * RT, 16)], aout.at[pl.ds(16 * RT, 16)]))

        pl.when(c == 0)(lambda: work(a0, 0))
        pl.when(c == 1)(lambda: work(a1, NCC))

    out = (jax.ShapeDtypeStruct((_N, _D), jnp.float32),
           jax.ShapeDtypeStruct((_N, _D), jnp.float32))
    return pl.kernel(
        body, out_type=out, mesh=mesh,
        scratch_types=[pltpu.VMEM_SHARED((_N, _D), jnp.float32)]
        + [pltpu.VMEM((_CH,), jnp.int32)] * (2 * NBUF)
        + [pltpu.VMEM((_CH, _D), jnp.float32)] * NBUF
        + [pltpu.VMEM((RZ, _D), jnp.float32)]
        + [pltpu.SemaphoreType.DMA] * (2 * NBUF),
        interpret=interpret)


@functools.lru_cache(maxsize=None)
def _conv_call(nin, nout, dr, interpret=False):
    NB = _N // _R

    def body(*refs):
        xs = refs[:nin]
        ps = refs[nin:3 * nin]
        w1, b1, w2, b2 = refs[3 * nin:3 * nin + 4]
        outs = refs[3 * nin + 4:3 * nin + 4 + nout]
        st = refs[3 * nin + 4 + nout]
        i = pl.program_id(0)

        h = b1[...]
        for k in range(nin):
            z = xs[k][...] + ps[2 * k][...] + ps[2 * k + 1][...]
            h = h + _dot(z, w1[128 * k:128 * (k + 1), :])
        h = jnp.maximum(h, 0.0)
        h = _dot(h, w2[...]) + b2[...]
        h = jnp.maximum(h, 0.0)
        for k in range(nout):
            outs[k][...] = h[:, 128 * k:128 * (k + 1)]

        @pl.when(i == 0)
        def _():
            st[...] = jnp.zeros_like(st)
        s1 = jnp.sum(h, axis=0)[None, :]
        s2 = jnp.sum(h * h, axis=0)[None, :]
        st[...] += jnp.concatenate([s1, s2], axis=0)

    slab = pl.BlockSpec((_R, _D), lambda i: (i, 0))
    return pl.pallas_call(
        body,
        grid=(NB,),
        in_specs=[slab] * (3 * nin) + [
            pl.BlockSpec((128 * nin, dr), lambda i: (0, 0)),
            pl.BlockSpec((1, dr), lambda i: (0, 0)),
            pl.BlockSpec((dr, 128 * nout), lambda i: (0, 0)),
            pl.BlockSpec((1, 128 * nout), lambda i: (0, 0)),
        ],
        out_specs=[slab] * nout + [pl.BlockSpec((2, 128 * nout),
                                                lambda i: (0, 0))],
        out_shape=[jax.ShapeDtypeStruct((_N, _D), jnp.float32)] * nout
        + [jax.ShapeDtypeStruct((2, 128 * nout), jnp.float32)],
        interpret=interpret)


@functools.lru_cache(maxsize=None)
def _bn_call(nout, interpret=False):
    NB = _N // _R

    def body(*refs):
        hs = refs[:nout]
        stref, g, b = refs[nout:nout + 3]
        outs = refs[nout + 3:]
        st = stref[...]
        mean = st[0:1, :] / _N
        var = st[1:2, :] / _N - mean * mean
        scale = g[...] * jax.lax.rsqrt(var + _EPS)
        shift = b[...] - mean * scale
        for k in range(nout):
            sl = slice(128 * k, 128 * (k + 1))
            outs[k][...] = hs[k][...] * scale[:, sl] + shift[:, sl]

    slab = pl.BlockSpec((_R, _D), lambda i: (i, 0))
    wide = pl.BlockSpec((2, 128 * nout), lambda i: (0, 0))
    row = pl.BlockSpec((1, 128 * nout), lambda i: (0, 0))
    return pl.pallas_call(
        body,
        grid=(NB,),
        in_specs=[slab] * nout + [wide, row, row],
        out_specs=[slab] * nout,
        out_shape=[jax.ShapeDtypeStruct((_N, _D), jnp.float32)] * nout,
        interpret=interpret)


@functools.lru_cache(maxsize=None)
def _pool_call(interpret=False):
    NB = _N // _R

    def body(h, bref, w1, b1, w2, b2, out, acc):
        i = pl.program_id(0)

        @pl.when(i == 0)
        def _():
            acc[...] = jnp.zeros_like(acc)
        ids = lax.broadcasted_iota(jnp.int32, (_NG, _R), 0)
        oht = (ids == bref[...].reshape(1, _R)).astype(jnp.float32)
        acc[...] += _dot(oht, h[...][:, :64])

        @pl.when(i == NB - 1)
        def _():
            g = jnp.maximum(_dot(acc[...], w1[...]) + b1[...], 0.0)
            g = jnp.maximum(_dot(g, w2[...]) + b2[...], 0.0)
            out[...] = g

    return pl.pallas_call(
        body,
        grid=(NB,),
        in_specs=[
            pl.BlockSpec((_R, _D), lambda i: (i, 0)),
            pl.BlockSpec((1, 1, _R), lambda i: (i, 0, 0)),
            pl.BlockSpec((64, 64), lambda i: (0, 0)),
            pl.BlockSpec((1, 64), lambda i: (0, 0)),
            pl.BlockSpec((64, 1), lambda i: (0, 0)),
            pl.BlockSpec((1, 1), lambda i: (0, 0)),
        ],
        out_specs=pl.BlockSpec((_NG, 1), lambda i: (0, 0)),
        out_shape=jax.ShapeDtypeStruct((_NG, 1), jnp.float32),
        scratch_shapes=[pltpu.VMEM((_NG, 64), jnp.float32)],
        interpret=interpret)


def _pad_cols(a, w):
    return a if a.shape[-1] == w else jnp.pad(a, [(0, 0)] * (a.ndim - 1)
                                              + [(0, w - a.shape[-1])])


def kernel(x, edge_index, edge_attr, batch, params):
    del edge_attr
    src = edge_index[0]
    dst = edge_index[1]
    batch3 = batch.reshape(_N // _R, 1, _R)
    slabs = [x]

    for convs, bn in zip(params["gins"], params["bns"]):
        nout = 1
        for p in convs:
            din, dout = p["W1"].shape
            nin = len(slabs)
            nout = 2 if dout == 256 else 1
            parts = []
            for sl in slabs:
                parts.extend(_agg_call()(sl, src, dst))
            w1p = jnp.pad(p["W1"], ((0, 128 * nin - din), (0, 0)))
            w2p = _pad_cols(p["W2"], 128 * nout)
            b2p = _pad_cols(p["b2"].reshape(1, dout), 128 * nout)
            res = _conv_call(nin, nout, dout)(
                *slabs, *parts, w1p, p["b1"].reshape(1, dout), w2p, b2p)
            slabs, stats = list(res[:nout]), res[nout]
        gp = _pad_cols(bn["gamma"].reshape(1, -1), 128 * nout)
        bp = _pad_cols(bn["beta"].reshape(1, -1), 128 * nout)
        slabs = list(_bn_call(nout)(*slabs, stats, gp, bp))

    fc1, fc2 = params["fc"]
    return _pool_call()(slabs[0], batch3,
                        fc1["W"], fc1["b"].reshape(1, 64),
                        fc2["W"], fc2["b"].reshape(1, 1))
